# Initial kernel scaffold; baseline (speedup 1.0000x reference)
import jax
import jax.numpy as jnp
from jax.experimental import pallas as pl


def kernel(x, edge_index, deg, W1, b1, W2, b2, edge_emb, att_w, gamma, beta):
    raise NotImplementedError("write your pallas kernel here")


# speedup vs baseline: 18.6723x; 18.6723x over previous
import jax
import jax.numpy as jnp
from jax import lax
from jax.experimental import pallas as pl
from jax.experimental.pallas import tpu as pltpu
from jax.experimental.pallas import tpu_sc as plsc

N = 10000
E = 320000
ATT_DIM = 128
H1 = 96

NSC = 2
NTILE = 16
EPT = E // NTILE
C = 78
BN = C * N
NBUCKET = -(-N // C)
B_PER_CORE = 65
HSZ = (NBUCKET + 2) * 16
DUMPBASE = BN
NSLOT = BN + 512
CHUNK = 256
LSZ = EPT + HSZ + CHUNK
VSZ = EPT + CHUNK
NP = 10240


def _scores_body(x_ref, w1_ref, b1_ref, w2t_ref, b2_ref, attw_ref, eemb_ref,
                 scores_ref, aux_ref):
  h = jnp.dot(x_ref[...], w1_ref[...], preferred_element_type=jnp.float32)
  h = h + b1_ref[...]
  h = jnp.where(h >= 0.0, h, 0.2 * h)
  sc = jnp.sum(h * w2t_ref[...], axis=1, keepdims=True) + b2_ref[0, 0]
  scores_ref[...] = sc
  spart = jnp.sum(sc)
  lanei = lax.broadcasted_iota(jnp.int32, (1, 16), 1)
  i = pl.program_id(0)

  @pl.when(i == 0)
  def _():
    a = attw_ref[0, 0]
    b = attw_ref[0, 21]
    cc = jnp.sum(attw_ref[0, 1:21] * eemb_ref[0, :])
    aux_ref[...] = (jnp.where(lanei == 0, spart, 0.0)
                    + jnp.where(lanei == 1, a, 0.0)
                    + jnp.where(lanei == 2, b, 0.0)
                    + jnp.where(lanei == 3, cc, 0.0))

  @pl.when(i > 0)
  def _():
    aux_ref[...] = aux_ref[...] + jnp.where(lanei == 0, spart, 0.0)


def _scores_call(x, w1, b1r, w2t, b2r, attwt, eemb):
  grid = 5
  r = N // grid
  return pl.pallas_call(
      _scores_body,
      grid=(grid,),
      in_specs=[
          pl.BlockSpec((r, ATT_DIM), lambda i: (i, 0)),
          pl.BlockSpec((ATT_DIM, H1), lambda i: (0, 0)),
          pl.BlockSpec((1, H1), lambda i: (0, 0)),
          pl.BlockSpec((1, H1), lambda i: (0, 0)),
          pl.BlockSpec((1, 1), lambda i: (0, 0)),
          pl.BlockSpec((1, 22), lambda i: (0, 0)),
          pl.BlockSpec((1, 20), lambda i: (0, 0)),
      ],
      out_specs=[
          pl.BlockSpec((r, 1), lambda i: (i, 0)),
          pl.BlockSpec((1, 16), lambda i: (0, 0)),
      ],
      out_shape=[
          jax.ShapeDtypeStruct((N, 1), jnp.float32),
          jax.ShapeDtypeStruct((1, 16), jnp.float32),
      ],
  )(x, w1, b1r, w2t, b2r, attwt, eemb)


def _edge_body(src_hbm, dst_hbm, scores_hbm, aux_hbm,
               num_hbm, den_hbm,
               sc_scores, key_l, sorted_k, v_l, bases, ctr, aux_v,
               zeros_c, ones_c, vm_c, srci_c, wnum_c, wden_c, att_c, tail_s,
               table, num_s, den_s):
  c = lax.axis_index("c")
  s = lax.axis_index("s")
  lane = lax.iota(jnp.int32, 16)
  lane15 = jnp.full((16, 1), 15, jnp.int32)
  gdn = lax.GatherDimensionNumbers(offset_dims=(), collapsed_slice_dims=(0,),
                                   start_index_map=(0,))

  def _splat_last(v):
    return lax.gather(v, lane15, gdn, (1,),
                      mode=lax.GatherScatterMode.PROMISE_IN_BOUNDS)

  pltpu.sync_copy(scores_hbm, sc_scores)
  pltpu.sync_copy(aux_hbm, aux_v)
  auxvec = aux_v[pl.ds(0, 16)]
  a_c = auxvec[1]
  b_c = auxvec[2]
  c_c = auxvec[3]

  base_e = s * EPT
  pltpu.sync_copy(src_hbm.at[pl.ds(base_e, EPT)], key_l)
  pltpu.sync_copy(dst_hbm.at[pl.ds(base_e, EPT)], sorted_k.at[pl.ds(0, EPT)])

  def _prep(i, _):
    sl = pl.ds(i * 16, 16)
    key_l[sl] = sorted_k[sl] * N + key_l[sl]
    return 0

  lax.fori_loop(0, EPT // 16, _prep, 0, unroll=False)

  def _zsmall(i, _):
    sl = pl.ds(i * 16, 16)
    zeros_c[sl] = jnp.zeros((16,), jnp.float32)
    ones_c[sl] = jnp.ones((16,), jnp.float32)
    return 0

  lax.fori_loop(0, CHUNK // 16, _zsmall, 0, unroll=False)

  def _zh(i, _):
    ctr[pl.ds(i * 16, 16)] = jnp.zeros((16,), jnp.int32)
    return 0

  lax.fori_loop(0, HSZ // 16, _zh, 0, unroll=False)

  def _hist(i, _):
    sl = pl.ds(i * 16, 16)
    cell = (key_l[sl] // BN) * 16 + lane
    old = plsc.load_gather(ctr, [cell])
    plsc.store_scatter(ctr, [cell], old + 1)
    return 0

  lax.fori_loop(0, EPT // 16, _hist, 0, unroll=False)

  def _prefix(bb, carry):
    carry = (carry + 15) & jnp.int32(-16)
    sl = pl.ds(bb * 16, 16)
    h = ctr[sl]
    incl = plsc.cumsum(h)
    excl = carry + incl - h
    bases[sl] = excl
    ctr[sl] = excl
    return carry + _splat_last(incl)

  lax.fori_loop(0, HSZ // 16, _prefix, jnp.zeros((16,), jnp.int32),
                unroll=False)

  def _scat(i, _):
    sl = pl.ds(i * 16, 16)
    kv = key_l[sl]
    cell = (kv // BN) * 16 + lane
    pos = plsc.load_gather(ctr, [cell])
    plsc.store_scatter(ctr, [cell], pos + 1)
    plsc.store_scatter(sorted_k, [pos], kv)
    return 0

  lax.fori_loop(0, EPT // 16, _scat, 0, unroll=False)

  @pl.when(s == 0)
  def _():
    def _znd(k, _):
      pltpu.sync_copy(zeros_c, num_s.at[pl.ds(k * CHUNK, CHUNK)])
      pltpu.sync_copy(zeros_c, den_s.at[pl.ds(k * CHUNK, CHUNK)])
      return 0
    lax.fori_loop(0, NP // CHUNK, _znd, 0, unroll=False)

  plsc.subcore_barrier()

  def _bucket(b_i, _):
    bkt = c * B_PER_CORE + b_i
    klo = bkt * BN
    base_dst = bkt * C
    start = pl.multiple_of(bases[pl.ds(bkt * 16, 16)][0], 16)
    end = ctr[pl.ds(bkt * 16, 16)][15]
    cnt = end - start
    nfull = cnt // CHUNK
    rem = cnt - nfull * CHUNK

    def _xf(j, _):
      base = start + j * 16
      gidx = base + lane
      m = gidx < end
      kv = sorted_k[pl.ds(base, 16)]
      plsc.store_scatter(sorted_k, [gidx], kv - klo, mask=m)
      return 0

    lax.fori_loop(0, (cnt + 15) // 16, _xf, 0, unroll=False)

    @pl.when(rem > 0)
    def _():
      def _tl(j, _):
        p = j * 16 + lane
        v = sorted_k[pl.ds(start + nfull * CHUNK + j * 16, 16)]
        valid = (nfull * CHUNK + p) < cnt
        tail_s[pl.ds(j * 16, 16)] = jnp.where(valid, v, DUMPBASE + p)
        return 0
      lax.fori_loop(0, CHUNK // 16, _tl, 0, unroll=False)

    def _att_into(slot):
      isdump = slot >= DUMPBASE
      srcv = slot % N
      dstv = jnp.where(isdump, 0, base_dst + slot // N)
      ssrc = plsc.load_gather(sc_scores, [srcv])
      sdst = plsc.load_gather(sc_scores, [dstv])
      return jnp.where(isdump, 0.0, a_c * ssrc + b_c * sdst + c_c)

    def _p1(k, _):
      pltpu.sync_copy(zeros_c, table.at[sorted_k.at[pl.ds(start + k * CHUNK,
                                                          CHUNK)]])
      return 0
    lax.fori_loop(0, nfull, _p1, 0, unroll=False)

    @pl.when(rem > 0)
    def _():
      pltpu.sync_copy(zeros_c, table.at[tail_s])
    plsc.subcore_barrier()

    def _p2(k, _):
      def _a(j, _):
        att_c[pl.ds(j * 16, 16)] = _att_into(
            sorted_k[pl.ds(start + k * CHUNK + j * 16, 16)])
        return 0
      lax.fori_loop(0, CHUNK // 16, _a, 0, unroll=False)
      pltpu.sync_copy(att_c, table.at[sorted_k.at[pl.ds(start + k * CHUNK,
                                                        CHUNK)]], add=True)
      return 0
    lax.fori_loop(0, nfull, _p2, 0, unroll=False)

    @pl.when(rem > 0)
    def _():
      def _a(j, _):
        att_c[pl.ds(j * 16, 16)] = _att_into(tail_s[pl.ds(j * 16, 16)])
        return 0
      lax.fori_loop(0, CHUNK // 16, _a, 0, unroll=False)
      pltpu.sync_copy(att_c, table.at[tail_s], add=True)
    plsc.subcore_barrier()

    def _p3(k, _):
      pltpu.sync_copy(table.at[sorted_k.at[pl.ds(start + k * CHUNK, CHUNK)]],
                      v_l.at[pl.ds(k * CHUNK, CHUNK)])
      return 0
    lax.fori_loop(0, nfull, _p3, 0, unroll=False)

    @pl.when(rem > 0)
    def _():
      pltpu.sync_copy(table.at[tail_s], v_l.at[pl.ds(nfull * CHUNK, CHUNK)])
    plsc.subcore_barrier()

    def _p4(k, _):
      pltpu.sync_copy(ones_c, table.at[sorted_k.at[pl.ds(start + k * CHUNK,
                                                         CHUNK)]], add=True)
      return 0
    lax.fori_loop(0, nfull, _p4, 0, unroll=False)

    @pl.when(rem > 0)
    def _():
      pltpu.sync_copy(ones_c, table.at[tail_s], add=True)
    plsc.subcore_barrier()

    def _p56_chunk(k, is_tail):
      if is_tail:
        pltpu.sync_copy(table.at[tail_s], vm_c)
      else:
        pltpu.sync_copy(table.at[sorted_k.at[pl.ds(start + k * CHUNK, CHUNK)]],
                        vm_c)

      def _inner(j, _):
        slc = pl.ds(j * 16, 16)
        if is_tail:
          slot = tail_s[slc]
          v = v_l[pl.ds(nfull * CHUNK + j * 16, 16)]
        else:
          slot = sorted_k[pl.ds(start + k * CHUNK + j * 16, 16)]
          v = v_l[pl.ds(k * CHUNK + j * 16, 16)]
        vm = vm_c[slc]
        mult = jnp.maximum((vm - v + 0.5).astype(jnp.int32)
                           .astype(jnp.float32), 1.0)
        lrel = jnp.where(v >= 0.0, v, 0.2 * v)
        w = (jnp.exp(lrel) - 1.0) / mult
        isdump = slot >= DUMPBASE
        srcv = slot % N
        dstv = jnp.where(isdump, 0, base_dst + slot // N)
        w = jnp.where(isdump, 0.0, w)
        sdst = plsc.load_gather(sc_scores, [dstv])
        srci_c[slc] = jnp.where(isdump, N + (slot - DUMPBASE) % 128, srcv)
        wnum_c[slc] = w * sdst
        wden_c[slc] = w
        return 0

      lax.fori_loop(0, CHUNK // 16, _inner, 0, unroll=False)
      pltpu.sync_copy(wnum_c, num_s.at[srci_c], add=True)
      pltpu.sync_copy(wden_c, den_s.at[srci_c], add=True)

    def _p56(k, _):
      _p56_chunk(k, False)
      return 0
    lax.fori_loop(0, nfull, _p56, 0, unroll=False)

    @pl.when(rem > 0)
    def _():
      _p56_chunk(0, True)
    plsc.subcore_barrier()
    return 0

  lax.fori_loop(0, B_PER_CORE, _bucket, 0, unroll=False)

  plsc.subcore_barrier()

  @pl.when(s == 0)
  def _():
    pltpu.sync_copy(num_s, num_hbm.at[c])
    pltpu.sync_copy(den_s, den_hbm.at[c])


def _edge_call(src, dst, scores, aux):
  mesh = plsc.VectorSubcoreMesh(core_axis_name="c", subcore_axis_name="s")
  f = pl.kernel(
      _edge_body,
      out_type=[
          jax.ShapeDtypeStruct((NSC, NP), jnp.float32),
          jax.ShapeDtypeStruct((NSC, NP), jnp.float32),
      ],
      mesh=mesh,
      compiler_params=pltpu.CompilerParams(use_tc_tiling_on_sc=False,
                                           needs_layout_passes=False),
      scratch_types=[
          pltpu.VMEM((N,), jnp.float32),
          pltpu.VMEM((EPT,), jnp.int32),
          pltpu.VMEM((LSZ,), jnp.int32),
          pltpu.VMEM((VSZ,), jnp.float32),
          pltpu.VMEM((HSZ,), jnp.int32),
          pltpu.VMEM((HSZ,), jnp.int32),
          pltpu.VMEM((16,), jnp.float32),
          pltpu.VMEM((CHUNK,), jnp.float32),
          pltpu.VMEM((CHUNK,), jnp.float32),
          pltpu.VMEM((CHUNK,), jnp.float32),
          pltpu.VMEM((CHUNK,), jnp.int32),
          pltpu.VMEM((CHUNK,), jnp.float32),
          pltpu.VMEM((CHUNK,), jnp.float32),
          pltpu.VMEM((CHUNK,), jnp.float32),
          pltpu.VMEM((CHUNK,), jnp.int32),
          pltpu.VMEM_SHARED((NSLOT,), jnp.float32),
          pltpu.VMEM_SHARED((NP,), jnp.float32),
          pltpu.VMEM_SHARED((NP,), jnp.float32),
      ],
  )
  return f(src, dst, scores, aux)


def _final_body(num_ref, den_ref, aux_ref, out_ref):
  s_tot = aux_ref[0, 0]
  nsum = num_ref[0, :] + num_ref[1, :]
  dsum = den_ref[0, :] + den_ref[1, :]
  res = (s_tot + nsum) / (jnp.float32(N) + dsum)
  out_ref[...] = res[:, None]


def _final_call(num, den, aux):
  grid = 5
  r = NP // grid
  return pl.pallas_call(
      _final_body,
      grid=(grid,),
      in_specs=[
          pl.BlockSpec((2, r), lambda i: (0, i)),
          pl.BlockSpec((2, r), lambda i: (0, i)),
          pl.BlockSpec((1, 16), lambda i: (0, 0)),
      ],
      out_specs=pl.BlockSpec((r, 1), lambda i: (i, 0)),
      out_shape=jax.ShapeDtypeStruct((NP, 1), jnp.float32),
  )(num, den, aux)


@jax.jit
def kernel(x, edge_index, deg, W1, b1, W2, b2, edge_emb, att_w, gamma, beta):
  del deg, gamma, beta
  b1r = b1.reshape(1, H1)
  w2t = W2.reshape(1, H1)
  b2r = b2.reshape(1, 1)
  attwt = att_w.reshape(1, 22)
  scores2d, aux = _scores_call(x, W1, b1r, w2t, b2r, attwt, edge_emb)
  scores = scores2d.reshape(N)
  num, den = _edge_call(edge_index[0], edge_index[1], scores,
                        aux.reshape(16))
  return _final_call(num, den, aux)[:N]


# speedup vs baseline: 19.1650x; 1.0264x over previous
import jax
import jax.numpy as jnp
from jax import lax
from jax.experimental import pallas as pl
from jax.experimental.pallas import tpu as pltpu
from jax.experimental.pallas import tpu_sc as plsc

N = 10000
E = 320000
ATT_DIM = 128
H1 = 96

NSC = 2
NTILE = 16
EPT = E // NTILE
C = 104
BN = C * N
NBUCKET = -(-N // C)
B_PER_CORE = 49
HSZ = (NBUCKET + 2) * 16
DUMPBASE = BN
NSLOT = BN + 512
CHUNK = 256
SB = 2000
LSZ = EPT + HSZ + CHUNK
VSZ = EPT + CHUNK
NP = 10240


def _scores_body(x_ref, w1_ref, b1_ref, w2t_ref, b2_ref, attw_ref, eemb_ref,
                 scores_ref, aux_ref):
  h = jnp.dot(x_ref[...], w1_ref[...], preferred_element_type=jnp.float32)
  h = h + b1_ref[...]
  h = jnp.where(h >= 0.0, h, 0.2 * h)
  sc = jnp.sum(h * w2t_ref[...], axis=1, keepdims=True) + b2_ref[0, 0]
  scores_ref[...] = sc
  spart = jnp.sum(sc)
  lanei = lax.broadcasted_iota(jnp.int32, (1, 16), 1)
  i = pl.program_id(0)

  @pl.when(i == 0)
  def _():
    a = attw_ref[0, 0]
    b = attw_ref[0, 21]
    cc = jnp.sum(attw_ref[0, 1:21] * eemb_ref[0, :])
    aux_ref[...] = (jnp.where(lanei == 0, spart, 0.0)
                    + jnp.where(lanei == 1, a, 0.0)
                    + jnp.where(lanei == 2, b, 0.0)
                    + jnp.where(lanei == 3, cc, 0.0))

  @pl.when(i > 0)
  def _():
    aux_ref[...] = aux_ref[...] + jnp.where(lanei == 0, spart, 0.0)


def _scores_call(x, w1, b1r, w2t, b2r, attwt, eemb):
  grid = 5
  r = N // grid
  return pl.pallas_call(
      _scores_body,
      grid=(grid,),
      in_specs=[
          pl.BlockSpec((r, ATT_DIM), lambda i: (i, 0)),
          pl.BlockSpec((ATT_DIM, H1), lambda i: (0, 0)),
          pl.BlockSpec((1, H1), lambda i: (0, 0)),
          pl.BlockSpec((1, H1), lambda i: (0, 0)),
          pl.BlockSpec((1, 1), lambda i: (0, 0)),
          pl.BlockSpec((1, 22), lambda i: (0, 0)),
          pl.BlockSpec((1, 20), lambda i: (0, 0)),
      ],
      out_specs=[
          pl.BlockSpec((r, 1), lambda i: (i, 0)),
          pl.BlockSpec((1, 16), lambda i: (0, 0)),
      ],
      out_shape=[
          jax.ShapeDtypeStruct((N, 1), jnp.float32),
          jax.ShapeDtypeStruct((1, 16), jnp.float32),
      ],
  )(x, w1, b1r, w2t, b2r, attwt, eemb)


def _edge_body(src_hbm, dst_hbm, scores_hbm, aux_hbm,
               num_hbm, den_hbm,
               sc_scores, sbuf, dbuf, sorted_k, v_l, bases, ctr, aux_v,
               zeros_c, ones_c, vm_c, srci_c, wnum_c, wden_c, att_c, tail_s,
               table, num_s, den_s):
  c = lax.axis_index("c")
  s = lax.axis_index("s")
  lane = lax.iota(jnp.int32, 16)
  lane15 = jnp.full((16, 1), 15, jnp.int32)
  gdn = lax.GatherDimensionNumbers(offset_dims=(), collapsed_slice_dims=(0,),
                                   start_index_map=(0,))

  def _splat_last(v):
    return lax.gather(v, lane15, gdn, (1,),
                      mode=lax.GatherScatterMode.PROMISE_IN_BOUNDS)

  pltpu.sync_copy(scores_hbm, sc_scores)
  pltpu.sync_copy(aux_hbm, aux_v)
  auxvec = aux_v[pl.ds(0, 16)]
  a_c = auxvec[1]
  b_c = auxvec[2]
  c_c = auxvec[3]

  base_e = s * EPT

  def _zsmall(i, _):
    sl = pl.ds(i * 16, 16)
    zeros_c[sl] = jnp.zeros((16,), jnp.float32)
    ones_c[sl] = jnp.ones((16,), jnp.float32)
    return 0

  lax.fori_loop(0, CHUNK // 16, _zsmall, 0, unroll=False)

  def _zh(i, _):
    ctr[pl.ds(i * 16, 16)] = jnp.zeros((16,), jnp.int32)
    return 0

  lax.fori_loop(0, HSZ // 16, _zh, 0, unroll=False)

  def _histc(cb, _):
    pltpu.sync_copy(dst_hbm.at[pl.ds(base_e + cb * SB, SB)], dbuf)

    def _hist(i, _):
      sl = pl.ds(i * 16, 16)
      cell = (dbuf[sl] // C) * 16 + lane
      old = plsc.load_gather(ctr, [cell])
      plsc.store_scatter(ctr, [cell], old + 1)
      return 0

    lax.fori_loop(0, SB // 16, _hist, 0, unroll=False)
    return 0

  lax.fori_loop(0, EPT // SB, _histc, 0, unroll=False)

  def _prefix(bb, carry):
    carry = (carry + 15) & jnp.int32(-16)
    sl = pl.ds(bb * 16, 16)
    h = ctr[sl]
    incl = plsc.cumsum(h)
    excl = carry + incl - h
    bases[sl] = excl
    ctr[sl] = excl
    return carry + _splat_last(incl)

  lax.fori_loop(0, HSZ // 16, _prefix, jnp.zeros((16,), jnp.int32),
                unroll=False)

  def _scatc(cb, _):
    pltpu.sync_copy(src_hbm.at[pl.ds(base_e + cb * SB, SB)], sbuf)
    pltpu.sync_copy(dst_hbm.at[pl.ds(base_e + cb * SB, SB)], dbuf)

    def _scat(i, _):
      sl = pl.ds(i * 16, 16)
      sv = sbuf[sl]
      dv = dbuf[sl]
      b = dv // C
      cell = b * 16 + lane
      pos = plsc.load_gather(ctr, [cell])
      plsc.store_scatter(ctr, [cell], pos + 1)
      plsc.store_scatter(sorted_k, [pos], (dv - b * C) * N + sv)
      return 0

    lax.fori_loop(0, SB // 16, _scat, 0, unroll=False)
    return 0

  lax.fori_loop(0, EPT // SB, _scatc, 0, unroll=False)

  @pl.when(s == 0)
  def _():
    def _znd(k, _):
      pltpu.sync_copy(zeros_c, num_s.at[pl.ds(k * CHUNK, CHUNK)])
      pltpu.sync_copy(zeros_c, den_s.at[pl.ds(k * CHUNK, CHUNK)])
      return 0
    lax.fori_loop(0, NP // CHUNK, _znd, 0, unroll=False)

  plsc.subcore_barrier()

  def _bucket(b_i, _):
    bkt = c * B_PER_CORE + b_i
    base_dst = bkt * C
    start = pl.multiple_of(bases[pl.ds(bkt * 16, 16)][0], 16)
    end = ctr[pl.ds(bkt * 16, 16)][15]
    cnt = end - start
    nfull = cnt // CHUNK
    rem = cnt - nfull * CHUNK

    @pl.when(rem > 0)
    def _():
      def _tl(j, _):
        p = j * 16 + lane
        v = sorted_k[pl.ds(start + nfull * CHUNK + j * 16, 16)]
        valid = (nfull * CHUNK + p) < cnt
        tail_s[pl.ds(j * 16, 16)] = jnp.where(valid, v, DUMPBASE + p)
        return 0
      lax.fori_loop(0, CHUNK // 16, _tl, 0, unroll=False)

    def _att_into(slot):
      isdump = slot >= DUMPBASE
      srcv = slot % N
      dstv = jnp.where(isdump, 0, base_dst + slot // N)
      ssrc = plsc.load_gather(sc_scores, [srcv])
      sdst = plsc.load_gather(sc_scores, [dstv])
      return jnp.where(isdump, 0.0, a_c * ssrc + b_c * sdst + c_c)

    def _p1(k, _):
      pltpu.sync_copy(zeros_c, table.at[sorted_k.at[pl.ds(start + k * CHUNK,
                                                          CHUNK)]])
      return 0
    lax.fori_loop(0, nfull, _p1, 0, unroll=False)

    @pl.when(rem > 0)
    def _():
      pltpu.sync_copy(zeros_c, table.at[tail_s])
    plsc.subcore_barrier()

    def _p2(k, _):
      def _a(j, _):
        att_c[pl.ds(j * 16, 16)] = _att_into(
            sorted_k[pl.ds(start + k * CHUNK + j * 16, 16)])
        return 0
      lax.fori_loop(0, CHUNK // 16, _a, 0, unroll=False)
      pltpu.sync_copy(att_c, table.at[sorted_k.at[pl.ds(start + k * CHUNK,
                                                        CHUNK)]], add=True)
      return 0
    lax.fori_loop(0, nfull, _p2, 0, unroll=False)

    @pl.when(rem > 0)
    def _():
      def _a(j, _):
        att_c[pl.ds(j * 16, 16)] = _att_into(tail_s[pl.ds(j * 16, 16)])
        return 0
      lax.fori_loop(0, CHUNK // 16, _a, 0, unroll=False)
      pltpu.sync_copy(att_c, table.at[tail_s], add=True)
    plsc.subcore_barrier()

    def _p3(k, _):
      pltpu.sync_copy(table.at[sorted_k.at[pl.ds(start + k * CHUNK, CHUNK)]],
                      v_l.at[pl.ds(k * CHUNK, CHUNK)])
      return 0
    lax.fori_loop(0, nfull, _p3, 0, unroll=False)

    @pl.when(rem > 0)
    def _():
      pltpu.sync_copy(table.at[tail_s], v_l.at[pl.ds(nfull * CHUNK, CHUNK)])
    plsc.subcore_barrier()

    def _p4(k, _):
      pltpu.sync_copy(ones_c, table.at[sorted_k.at[pl.ds(start + k * CHUNK,
                                                         CHUNK)]], add=True)
      return 0
    lax.fori_loop(0, nfull, _p4, 0, unroll=False)

    @pl.when(rem > 0)
    def _():
      pltpu.sync_copy(ones_c, table.at[tail_s], add=True)
    plsc.subcore_barrier()

    def _p56_chunk(k, is_tail):
      if is_tail:
        pltpu.sync_copy(table.at[tail_s], vm_c)
      else:
        pltpu.sync_copy(table.at[sorted_k.at[pl.ds(start + k * CHUNK, CHUNK)]],
                        vm_c)

      def _inner(j, _):
        slc = pl.ds(j * 16, 16)
        if is_tail:
          slot = tail_s[slc]
          v = v_l[pl.ds(nfull * CHUNK + j * 16, 16)]
        else:
          slot = sorted_k[pl.ds(start + k * CHUNK + j * 16, 16)]
          v = v_l[pl.ds(k * CHUNK + j * 16, 16)]
        vm = vm_c[slc]
        mult = jnp.maximum((vm - v + 0.5).astype(jnp.int32)
                           .astype(jnp.float32), 1.0)
        lrel = jnp.where(v >= 0.0, v, 0.2 * v)
        w = (jnp.exp(lrel) - 1.0) / mult
        isdump = slot >= DUMPBASE
        srcv = slot % N
        dstv = jnp.where(isdump, 0, base_dst + slot // N)
        w = jnp.where(isdump, 0.0, w)
        sdst = plsc.load_gather(sc_scores, [dstv])
        srci_c[slc] = jnp.where(isdump, N + (slot - DUMPBASE) % 128, srcv)
        wnum_c[slc] = w * sdst
        wden_c[slc] = w
        return 0

      lax.fori_loop(0, CHUNK // 16, _inner, 0, unroll=False)
      pltpu.sync_copy(wnum_c, num_s.at[srci_c], add=True)
      pltpu.sync_copy(wden_c, den_s.at[srci_c], add=True)

    def _p56(k, _):
      _p56_chunk(k, False)
      return 0
    lax.fori_loop(0, nfull, _p56, 0, unroll=False)

    @pl.when(rem > 0)
    def _():
      _p56_chunk(0, True)
    plsc.subcore_barrier()
    return 0

  lax.fori_loop(0, B_PER_CORE, _bucket, 0, unroll=False)

  plsc.subcore_barrier()

  @pl.when(s == 0)
  def _():
    pltpu.sync_copy(num_s, num_hbm.at[c])
    pltpu.sync_copy(den_s, den_hbm.at[c])


def _edge_call(src, dst, scores, aux):
  mesh = plsc.VectorSubcoreMesh(core_axis_name="c", subcore_axis_name="s")
  f = pl.kernel(
      _edge_body,
      out_type=[
          jax.ShapeDtypeStruct((NSC, NP), jnp.float32),
          jax.ShapeDtypeStruct((NSC, NP), jnp.float32),
      ],
      mesh=mesh,
      compiler_params=pltpu.CompilerParams(use_tc_tiling_on_sc=False,
                                           needs_layout_passes=False),
      scratch_types=[
          pltpu.VMEM((N,), jnp.float32),
          pltpu.VMEM((SB,), jnp.int32),
          pltpu.VMEM((SB,), jnp.int32),
          pltpu.VMEM((LSZ,), jnp.int32),
          pltpu.VMEM((VSZ,), jnp.float32),
          pltpu.VMEM((HSZ,), jnp.int32),
          pltpu.VMEM((HSZ,), jnp.int32),
          pltpu.VMEM((16,), jnp.float32),
          pltpu.VMEM((CHUNK,), jnp.float32),
          pltpu.VMEM((CHUNK,), jnp.float32),
          pltpu.VMEM((CHUNK,), jnp.float32),
          pltpu.VMEM((CHUNK,), jnp.int32),
          pltpu.VMEM((CHUNK,), jnp.float32),
          pltpu.VMEM((CHUNK,), jnp.float32),
          pltpu.VMEM((CHUNK,), jnp.float32),
          pltpu.VMEM((CHUNK,), jnp.int32),
          pltpu.VMEM_SHARED((NSLOT,), jnp.float32),
          pltpu.VMEM_SHARED((NP,), jnp.float32),
          pltpu.VMEM_SHARED((NP,), jnp.float32),
      ],
  )
  return f(src, dst, scores, aux)


def _final_body(num_ref, den_ref, aux_ref, out_ref):
  s_tot = aux_ref[0, 0]
  nsum = num_ref[0, :] + num_ref[1, :]
  dsum = den_ref[0, :] + den_ref[1, :]
  res = (s_tot + nsum) / (jnp.float32(N) + dsum)
  out_ref[...] = res[:, None]


def _final_call(num, den, aux):
  grid = 5
  r = NP // grid
  return pl.pallas_call(
      _final_body,
      grid=(grid,),
      in_specs=[
          pl.BlockSpec((2, r), lambda i: (0, i)),
          pl.BlockSpec((2, r), lambda i: (0, i)),
          pl.BlockSpec((1, 16), lambda i: (0, 0)),
      ],
      out_specs=pl.BlockSpec((r, 1), lambda i: (i, 0)),
      out_shape=jax.ShapeDtypeStruct((NP, 1), jnp.float32),
  )(num, den, aux)


@jax.jit
def kernel(x, edge_index, deg, W1, b1, W2, b2, edge_emb, att_w, gamma, beta):
  del deg, gamma, beta
  b1r = b1.reshape(1, H1)
  w2t = W2.reshape(1, H1)
  b2r = b2.reshape(1, 1)
  attwt = att_w.reshape(1, 22)
  scores2d, aux = _scores_call(x, W1, b1r, w2t, b2r, attwt, edge_emb)
  scores = scores2d.reshape(N)
  num, den = _edge_call(edge_index[0], edge_index[1], scores,
                        aux.reshape(16))
  return _final_call(num, den, aux)[:N]


# speedup vs baseline: 19.3990x; 1.0122x over previous
import jax
import jax.numpy as jnp
from jax import lax
from jax.experimental import pallas as pl
from jax.experimental.pallas import tpu as pltpu
from jax.experimental.pallas import tpu_sc as plsc

N = 10000
E = 320000
ATT_DIM = 128
H1 = 96

NSC = 2
NTILE = 16
EPT = E // NTILE
C = 96
BN = C * N
NBUCKET = -(-N // C)
B_PER_CORE = 53
NU = 4
HSZ = (NBUCKET + 2) * 16
DUMPBASE = BN
NSLOT = BN + 512
CHUNK = 256
SB = 2000
LSZ = EPT + HSZ + CHUNK
VSZ = EPT + CHUNK
NP = 10240


def _scores_body(x_ref, w1_ref, b1_ref, w2t_ref, b2_ref, attw_ref, eemb_ref,
                 scores_ref, aux_ref):
  h = jnp.dot(x_ref[...], w1_ref[...], preferred_element_type=jnp.float32)
  h = h + b1_ref[...]
  h = jnp.where(h >= 0.0, h, 0.2 * h)
  sc = jnp.sum(h * w2t_ref[...], axis=1, keepdims=True) + b2_ref[0, 0]
  scores_ref[...] = sc
  spart = jnp.sum(sc)
  lanei = lax.broadcasted_iota(jnp.int32, (1, 16), 1)
  i = pl.program_id(0)

  @pl.when(i == 0)
  def _():
    a = attw_ref[0, 0]
    b = attw_ref[0, 21]
    cc = jnp.sum(attw_ref[0, 1:21] * eemb_ref[0, :])
    aux_ref[...] = (jnp.where(lanei == 0, spart, 0.0)
                    + jnp.where(lanei == 1, a, 0.0)
                    + jnp.where(lanei == 2, b, 0.0)
                    + jnp.where(lanei == 3, cc, 0.0))

  @pl.when(i > 0)
  def _():
    aux_ref[...] = aux_ref[...] + jnp.where(lanei == 0, spart, 0.0)


def _scores_call(x, w1, b1r, w2t, b2r, attwt, eemb):
  grid = 5
  r = N // grid
  return pl.pallas_call(
      _scores_body,
      grid=(grid,),
      in_specs=[
          pl.BlockSpec((r, ATT_DIM), lambda i: (i, 0)),
          pl.BlockSpec((ATT_DIM, H1), lambda i: (0, 0)),
          pl.BlockSpec((1, H1), lambda i: (0, 0)),
          pl.BlockSpec((1, H1), lambda i: (0, 0)),
          pl.BlockSpec((1, 1), lambda i: (0, 0)),
          pl.BlockSpec((1, 22), lambda i: (0, 0)),
          pl.BlockSpec((1, 20), lambda i: (0, 0)),
      ],
      out_specs=[
          pl.BlockSpec((r, 1), lambda i: (i, 0)),
          pl.BlockSpec((1, 16), lambda i: (0, 0)),
      ],
      out_shape=[
          jax.ShapeDtypeStruct((N, 1), jnp.float32),
          jax.ShapeDtypeStruct((1, 16), jnp.float32),
      ],
  )(x, w1, b1r, w2t, b2r, attwt, eemb)


def _edge_body(src_hbm, dst_hbm, scores_hbm, aux_hbm,
               num_hbm, den_hbm,
               sc_scores, sbuf, dbuf, sorted_k, v_l, bases,
               ctr0, ctr1, ctr2, ctr3, aux_v,
               zeros_c, ones_c, vm_c, srci_c, wnum_c, wden_c, att_c, tail_s,
               table, num_s, den_s):
  c = lax.axis_index("c")
  s = lax.axis_index("s")
  lane = lax.iota(jnp.int32, 16)
  lane15 = jnp.full((16, 1), 15, jnp.int32)
  gdn = lax.GatherDimensionNumbers(offset_dims=(), collapsed_slice_dims=(0,),
                                   start_index_map=(0,))

  def _splat_last(v):
    return lax.gather(v, lane15, gdn, (1,),
                      mode=lax.GatherScatterMode.PROMISE_IN_BOUNDS)

  pltpu.sync_copy(scores_hbm, sc_scores)
  pltpu.sync_copy(aux_hbm, aux_v)
  auxvec = aux_v[pl.ds(0, 16)]
  a_c = auxvec[1]
  b_c = auxvec[2]
  c_c = auxvec[3]

  base_e = s * EPT

  def _zsmall(i, _):
    sl = pl.ds(i * 16, 16)
    zeros_c[sl] = jnp.zeros((16,), jnp.float32)
    ones_c[sl] = jnp.ones((16,), jnp.float32)
    return 0

  lax.fori_loop(0, CHUNK // 16, _zsmall, 0, unroll=False)

  ctrs = (ctr0, ctr1, ctr2, ctr3)

  def _zh(i, _):
    sl = pl.ds(i * 16, 16)
    z = jnp.zeros((16,), jnp.int32)
    ctr0[sl] = z
    ctr1[sl] = z
    ctr2[sl] = z
    ctr3[sl] = z
    return 0

  lax.fori_loop(0, HSZ // 16, _zh, 0, unroll=False)

  def _histc(cb, _):
    pltpu.sync_copy(dst_hbm.at[pl.ds(base_e + cb * SB, SB)], dbuf)

    def _hist(i, _):
      for u in range(NU):
        sl = pl.ds(i * (16 * NU) + u * 16, 16)
        cell = (dbuf[sl] // C) * 16 + lane
        old = plsc.load_gather(ctrs[u], [cell])
        plsc.store_scatter(ctrs[u], [cell], old + 1)
      return 0

    lax.fori_loop(0, SB // (16 * NU), _hist, 0, unroll=False)

    def _htail(i, _):
      sl = pl.ds((SB // (16 * NU)) * (16 * NU) + i * 16, 16)
      cell = (dbuf[sl] // C) * 16 + lane
      old = plsc.load_gather(ctr0, [cell])
      plsc.store_scatter(ctr0, [cell], old + 1)
      return 0

    lax.fori_loop(0, (SB % (16 * NU)) // 16, _htail, 0, unroll=False)
    return 0

  lax.fori_loop(0, EPT // SB, _histc, 0, unroll=False)

  def _prefix(bb, carry):
    carry = (carry + 15) & jnp.int32(-16)
    sl = pl.ds(bb * 16, 16)
    for u in range(NU):
      h = ctrs[u][sl]
      incl = plsc.cumsum(h)
      excl = carry + incl - h
      if u == 0:
        bases[sl] = excl
      ctrs[u][sl] = excl
      carry = carry + _splat_last(incl)
    return carry

  lax.fori_loop(0, HSZ // 16, _prefix, jnp.zeros((16,), jnp.int32),
                unroll=False)

  def _scatc(cb, _):
    pltpu.sync_copy(src_hbm.at[pl.ds(base_e + cb * SB, SB)], sbuf)
    pltpu.sync_copy(dst_hbm.at[pl.ds(base_e + cb * SB, SB)], dbuf)

    def _scat1(sl, cu):
      sv = sbuf[sl]
      dv = dbuf[sl]
      b = dv // C
      cell = b * 16 + lane
      pos = plsc.load_gather(cu, [cell])
      plsc.store_scatter(cu, [cell], pos + 1)
      plsc.store_scatter(sorted_k, [pos], (dv - b * C) * N + sv)

    def _scat(i, _):
      for u in range(NU):
        _scat1(pl.ds(i * (16 * NU) + u * 16, 16), ctrs[u])
      return 0

    lax.fori_loop(0, SB // (16 * NU), _scat, 0, unroll=False)

    def _stail(i, _):
      _scat1(pl.ds((SB // (16 * NU)) * (16 * NU) + i * 16, 16), ctr0)
      return 0

    lax.fori_loop(0, (SB % (16 * NU)) // 16, _stail, 0, unroll=False)
    return 0

  lax.fori_loop(0, EPT // SB, _scatc, 0, unroll=False)

  @pl.when(s == 0)
  def _():
    def _znd(k, _):
      pltpu.sync_copy(zeros_c, num_s.at[pl.ds(k * CHUNK, CHUNK)])
      pltpu.sync_copy(zeros_c, den_s.at[pl.ds(k * CHUNK, CHUNK)])
      return 0
    lax.fori_loop(0, NP // CHUNK, _znd, 0, unroll=False)

  plsc.subcore_barrier()

  def _bucket(b_i, _):
    bkt = c * B_PER_CORE + b_i
    base_dst = bkt * C
    start = pl.multiple_of(bases[pl.ds(bkt * 16, 16)][0], 16)
    end = ctr3[pl.ds(bkt * 16, 16)][15]
    cnt = end - start
    nfull = cnt // CHUNK
    rem = cnt - nfull * CHUNK

    @pl.when(rem > 0)
    def _():
      def _tl(j, _):
        p = j * 16 + lane
        v = sorted_k[pl.ds(start + nfull * CHUNK + j * 16, 16)]
        valid = (nfull * CHUNK + p) < cnt
        tail_s[pl.ds(j * 16, 16)] = jnp.where(valid, v, DUMPBASE + p)
        return 0
      lax.fori_loop(0, CHUNK // 16, _tl, 0, unroll=False)

    def _att_into(slot):
      isdump = slot >= DUMPBASE
      srcv = slot % N
      dstv = jnp.where(isdump, 0, base_dst + slot // N)
      ssrc = plsc.load_gather(sc_scores, [srcv])
      sdst = plsc.load_gather(sc_scores, [dstv])
      return jnp.where(isdump, 0.0, a_c * ssrc + b_c * sdst + c_c)

    def _p1(k, _):
      pltpu.sync_copy(zeros_c, table.at[sorted_k.at[pl.ds(start + k * CHUNK,
                                                          CHUNK)]])
      return 0
    lax.fori_loop(0, nfull, _p1, 0, unroll=False)

    @pl.when(rem > 0)
    def _():
      pltpu.sync_copy(zeros_c, table.at[tail_s])
    plsc.subcore_barrier()

    def _p2(k, _):
      def _a(j, _):
        att_c[pl.ds(j * 16, 16)] = _att_into(
            sorted_k[pl.ds(start + k * CHUNK + j * 16, 16)])
        return 0
      lax.fori_loop(0, CHUNK // 16, _a, 0, unroll=False)
      pltpu.sync_copy(att_c, table.at[sorted_k.at[pl.ds(start + k * CHUNK,
                                                        CHUNK)]], add=True)
      return 0
    lax.fori_loop(0, nfull, _p2, 0, unroll=False)

    @pl.when(rem > 0)
    def _():
      def _a(j, _):
        att_c[pl.ds(j * 16, 16)] = _att_into(tail_s[pl.ds(j * 16, 16)])
        return 0
      lax.fori_loop(0, CHUNK // 16, _a, 0, unroll=False)
      pltpu.sync_copy(att_c, table.at[tail_s], add=True)
    plsc.subcore_barrier()

    def _p3(k, _):
      pltpu.sync_copy(table.at[sorted_k.at[pl.ds(start + k * CHUNK, CHUNK)]],
                      v_l.at[pl.ds(k * CHUNK, CHUNK)])
      return 0
    lax.fori_loop(0, nfull, _p3, 0, unroll=False)

    @pl.when(rem > 0)
    def _():
      pltpu.sync_copy(table.at[tail_s], v_l.at[pl.ds(nfull * CHUNK, CHUNK)])
    plsc.subcore_barrier()

    def _p4(k, _):
      pltpu.sync_copy(ones_c, table.at[sorted_k.at[pl.ds(start + k * CHUNK,
                                                         CHUNK)]], add=True)
      return 0
    lax.fori_loop(0, nfull, _p4, 0, unroll=False)

    @pl.when(rem > 0)
    def _():
      pltpu.sync_copy(ones_c, table.at[tail_s], add=True)
    plsc.subcore_barrier()

    def _p56_chunk(k, is_tail):
      if is_tail:
        pltpu.sync_copy(table.at[tail_s], vm_c)
      else:
        pltpu.sync_copy(table.at[sorted_k.at[pl.ds(start + k * CHUNK, CHUNK)]],
                        vm_c)

      def _inner(j, _):
        slc = pl.ds(j * 16, 16)
        if is_tail:
          slot = tail_s[slc]
          v = v_l[pl.ds(nfull * CHUNK + j * 16, 16)]
        else:
          slot = sorted_k[pl.ds(start + k * CHUNK + j * 16, 16)]
          v = v_l[pl.ds(k * CHUNK + j * 16, 16)]
        vm = vm_c[slc]
        mult = jnp.maximum((vm - v + 0.5).astype(jnp.int32)
                           .astype(jnp.float32), 1.0)
        lrel = jnp.where(v >= 0.0, v, 0.2 * v)
        w = (jnp.exp(lrel) - 1.0) / mult
        isdump = slot >= DUMPBASE
        srcv = slot % N
        dstv = jnp.where(isdump, 0, base_dst + slot // N)
        w = jnp.where(isdump, 0.0, w)
        sdst = plsc.load_gather(sc_scores, [dstv])
        srci_c[slc] = jnp.where(isdump, N + (slot - DUMPBASE) % 128, srcv)
        wnum_c[slc] = w * sdst
        wden_c[slc] = w
        return 0

      lax.fori_loop(0, CHUNK // 16, _inner, 0, unroll=False)
      pltpu.sync_copy(wnum_c, num_s.at[srci_c], add=True)
      pltpu.sync_copy(wden_c, den_s.at[srci_c], add=True)

    def _p56(k, _):
      _p56_chunk(k, False)
      return 0
    lax.fori_loop(0, nfull, _p56, 0, unroll=False)

    @pl.when(rem > 0)
    def _():
      _p56_chunk(0, True)
    plsc.subcore_barrier()
    return 0

  lax.fori_loop(0, B_PER_CORE, _bucket, 0, unroll=False)

  plsc.subcore_barrier()

  @pl.when(s == 0)
  def _():
    pltpu.sync_copy(num_s, num_hbm.at[c])
    pltpu.sync_copy(den_s, den_hbm.at[c])


def _edge_call(src, dst, scores, aux):
  mesh = plsc.VectorSubcoreMesh(core_axis_name="c", subcore_axis_name="s")
  f = pl.kernel(
      _edge_body,
      out_type=[
          jax.ShapeDtypeStruct((NSC, NP), jnp.float32),
          jax.ShapeDtypeStruct((NSC, NP), jnp.float32),
      ],
      mesh=mesh,
      compiler_params=pltpu.CompilerParams(use_tc_tiling_on_sc=False,
                                           needs_layout_passes=False),
      scratch_types=[
          pltpu.VMEM((N,), jnp.float32),
          pltpu.VMEM((SB,), jnp.int32),
          pltpu.VMEM((SB,), jnp.int32),
          pltpu.VMEM((LSZ,), jnp.int32),
          pltpu.VMEM((VSZ,), jnp.float32),
          pltpu.VMEM((HSZ,), jnp.int32),
          pltpu.VMEM((HSZ,), jnp.int32),
          pltpu.VMEM((HSZ,), jnp.int32),
          pltpu.VMEM((HSZ,), jnp.int32),
          pltpu.VMEM((HSZ,), jnp.int32),
          pltpu.VMEM((16,), jnp.float32),
          pltpu.VMEM((CHUNK,), jnp.float32),
          pltpu.VMEM((CHUNK,), jnp.float32),
          pltpu.VMEM((CHUNK,), jnp.float32),
          pltpu.VMEM((CHUNK,), jnp.int32),
          pltpu.VMEM((CHUNK,), jnp.float32),
          pltpu.VMEM((CHUNK,), jnp.float32),
          pltpu.VMEM((CHUNK,), jnp.float32),
          pltpu.VMEM((CHUNK,), jnp.int32),
          pltpu.VMEM_SHARED((NSLOT,), jnp.float32),
          pltpu.VMEM_SHARED((NP,), jnp.float32),
          pltpu.VMEM_SHARED((NP,), jnp.float32),
      ],
  )
  return f(src, dst, scores, aux)


def _final_body(num_ref, den_ref, aux_ref, out_ref):
  s_tot = aux_ref[0, 0]
  nsum = num_ref[0, :] + num_ref[1, :]
  dsum = den_ref[0, :] + den_ref[1, :]
  res = (s_tot + nsum) / (jnp.float32(N) + dsum)
  out_ref[...] = res[:, None]


def _final_call(num, den, aux):
  grid = 5
  r = NP // grid
  return pl.pallas_call(
      _final_body,
      grid=(grid,),
      in_specs=[
          pl.BlockSpec((2, r), lambda i: (0, i)),
          pl.BlockSpec((2, r), lambda i: (0, i)),
          pl.BlockSpec((1, 16), lambda i: (0, 0)),
      ],
      out_specs=pl.BlockSpec((r, 1), lambda i: (i, 0)),
      out_shape=jax.ShapeDtypeStruct((NP, 1), jnp.float32),
  )(num, den, aux)


@jax.jit
def kernel(x, edge_index, deg, W1, b1, W2, b2, edge_emb, att_w, gamma, beta):
  del deg, gamma, beta
  b1r = b1.reshape(1, H1)
  w2t = W2.reshape(1, H1)
  b2r = b2.reshape(1, 1)
  attwt = att_w.reshape(1, 22)
  scores2d, aux = _scores_call(x, W1, b1r, w2t, b2r, attwt, edge_emb)
  scores = scores2d.reshape(N)
  num, den = _edge_call(edge_index[0], edge_index[1], scores,
                        aux.reshape(16))
  return _final_call(num, den, aux)[:N]


# speedup vs baseline: 19.8473x; 1.0231x over previous
import jax
import jax.numpy as jnp
from jax import lax
from jax.experimental import pallas as pl
from jax.experimental.pallas import tpu as pltpu
from jax.experimental.pallas import tpu_sc as plsc

N = 10000
E = 320000
ATT_DIM = 128
H1 = 96

NSC = 2
NTILE = 16
EPT = E // NTILE
C = 96
BN = C * N
NBUCKET = -(-N // C)
B_PER_CORE = 53
NU = 4
HSZ = (NBUCKET + 2) * 16
DUMPBASE = BN
NSLOT = BN + 512
CHUNK = 256
SB = 2000
LSZ = EPT + HSZ + CHUNK
VSZ = EPT + CHUNK
NP = 10240


def _scores_body(x_ref, w1_ref, b1_ref, w2t_ref, b2_ref, attw_ref, eemb_ref,
                 scores_ref, aux_ref):
  h = jnp.dot(x_ref[...], w1_ref[...], preferred_element_type=jnp.float32)
  h = h + b1_ref[...]
  h = jnp.where(h >= 0.0, h, 0.2 * h)
  sc = jnp.sum(h * w2t_ref[...], axis=1, keepdims=True) + b2_ref[0, 0]
  scores_ref[...] = sc
  spart = jnp.sum(sc)
  lanei = lax.broadcasted_iota(jnp.int32, (1, 16), 1)
  i = pl.program_id(0)

  @pl.when(i == 0)
  def _():
    a = attw_ref[0, 0]
    b = attw_ref[0, 21]
    cc = jnp.sum(attw_ref[0, 1:21] * eemb_ref[0, :])
    aux_ref[...] = (jnp.where(lanei == 0, spart, 0.0)
                    + jnp.where(lanei == 1, a, 0.0)
                    + jnp.where(lanei == 2, b, 0.0)
                    + jnp.where(lanei == 3, cc, 0.0))

  @pl.when(i > 0)
  def _():
    aux_ref[...] = aux_ref[...] + jnp.where(lanei == 0, spart, 0.0)


def _scores_call(x, w1, b1r, w2t, b2r, attwt, eemb):
  grid = 5
  r = N // grid
  return pl.pallas_call(
      _scores_body,
      grid=(grid,),
      in_specs=[
          pl.BlockSpec((r, ATT_DIM), lambda i: (i, 0)),
          pl.BlockSpec((ATT_DIM, H1), lambda i: (0, 0)),
          pl.BlockSpec((1, H1), lambda i: (0, 0)),
          pl.BlockSpec((1, H1), lambda i: (0, 0)),
          pl.BlockSpec((1, 1), lambda i: (0, 0)),
          pl.BlockSpec((1, 22), lambda i: (0, 0)),
          pl.BlockSpec((1, 20), lambda i: (0, 0)),
      ],
      out_specs=[
          pl.BlockSpec((r, 1), lambda i: (i, 0)),
          pl.BlockSpec((1, 16), lambda i: (0, 0)),
      ],
      out_shape=[
          jax.ShapeDtypeStruct((N, 1), jnp.float32),
          jax.ShapeDtypeStruct((1, 16), jnp.float32),
      ],
  )(x, w1, b1r, w2t, b2r, attwt, eemb)


def _edge_body(src_hbm, dst_hbm, scores_hbm, aux_hbm,
               num_hbm, den_hbm,
               sc_scores, sbuf, dbuf, sorted_k, v_l, bases,
               ctr0, ctr1, ctr2, ctr3, aux_v,
               zeros_c, ones_c, vm_c, srci_c, wnum_c, wden_c, att_c, tail_s,
               nd_sem1, nd_sem2,
               table, num_s, den_s):
  c = lax.axis_index("c")
  s = lax.axis_index("s")
  lane = lax.iota(jnp.int32, 16)
  lane15 = jnp.full((16, 1), 15, jnp.int32)
  gdn = lax.GatherDimensionNumbers(offset_dims=(), collapsed_slice_dims=(0,),
                                   start_index_map=(0,))

  def _splat_last(v):
    return lax.gather(v, lane15, gdn, (1,),
                      mode=lax.GatherScatterMode.PROMISE_IN_BOUNDS)

  pltpu.sync_copy(scores_hbm, sc_scores)
  pltpu.sync_copy(aux_hbm, aux_v)
  auxvec = aux_v[pl.ds(0, 16)]
  a_c = auxvec[1]
  b_c = auxvec[2]
  c_c = auxvec[3]

  base_e = s * EPT

  def _zsmall(i, _):
    sl = pl.ds(i * 16, 16)
    zeros_c[sl] = jnp.zeros((16,), jnp.float32)
    ones_c[sl] = jnp.ones((16,), jnp.float32)
    return 0

  lax.fori_loop(0, CHUNK // 16, _zsmall, 0, unroll=False)

  ctrs = (ctr0, ctr1, ctr2, ctr3)

  def _zh(i, _):
    sl = pl.ds(i * 16, 16)
    z = jnp.zeros((16,), jnp.int32)
    ctr0[sl] = z
    ctr1[sl] = z
    ctr2[sl] = z
    ctr3[sl] = z
    return 0

  lax.fori_loop(0, HSZ // 16, _zh, 0, unroll=False)

  def _histc(cb, _):
    pltpu.sync_copy(dst_hbm.at[pl.ds(base_e + cb * SB, SB)], dbuf)

    def _hist(i, _):
      for u in range(NU):
        sl = pl.ds(i * (16 * NU) + u * 16, 16)
        cell = (dbuf[sl] // C) * 16 + lane
        old = plsc.load_gather(ctrs[u], [cell])
        plsc.store_scatter(ctrs[u], [cell], old + 1)
      return 0

    lax.fori_loop(0, SB // (16 * NU), _hist, 0, unroll=False)

    def _htail(i, _):
      sl = pl.ds((SB // (16 * NU)) * (16 * NU) + i * 16, 16)
      cell = (dbuf[sl] // C) * 16 + lane
      old = plsc.load_gather(ctr0, [cell])
      plsc.store_scatter(ctr0, [cell], old + 1)
      return 0

    lax.fori_loop(0, (SB % (16 * NU)) // 16, _htail, 0, unroll=False)
    return 0

  lax.fori_loop(0, EPT // SB, _histc, 0, unroll=False)

  def _prefix(bb, carry):
    carry = (carry + 15) & jnp.int32(-16)
    sl = pl.ds(bb * 16, 16)
    for u in range(NU):
      h = ctrs[u][sl]
      incl = plsc.cumsum(h)
      excl = carry + incl - h
      if u == 0:
        bases[sl] = excl
      ctrs[u][sl] = excl
      carry = carry + _splat_last(incl)
    return carry

  lax.fori_loop(0, HSZ // 16, _prefix, jnp.zeros((16,), jnp.int32),
                unroll=False)

  def _scatc(cb, _):
    pltpu.sync_copy(src_hbm.at[pl.ds(base_e + cb * SB, SB)], sbuf)
    pltpu.sync_copy(dst_hbm.at[pl.ds(base_e + cb * SB, SB)], dbuf)

    def _scat1(sl, cu):
      sv = sbuf[sl]
      dv = dbuf[sl]
      b = dv // C
      cell = b * 16 + lane
      pos = plsc.load_gather(cu, [cell])
      plsc.store_scatter(cu, [cell], pos + 1)
      plsc.store_scatter(sorted_k, [pos], (dv - b * C) * N + sv)

    def _scat(i, _):
      for u in range(NU):
        _scat1(pl.ds(i * (16 * NU) + u * 16, 16), ctrs[u])
      return 0

    lax.fori_loop(0, SB // (16 * NU), _scat, 0, unroll=False)

    def _stail(i, _):
      _scat1(pl.ds((SB // (16 * NU)) * (16 * NU) + i * 16, 16), ctr0)
      return 0

    lax.fori_loop(0, (SB % (16 * NU)) // 16, _stail, 0, unroll=False)
    return 0

  lax.fori_loop(0, EPT // SB, _scatc, 0, unroll=False)

  def _znd(j, _):
    k = s + j * 16

    @pl.when(k < NP // CHUNK)
    def _():
      pltpu.sync_copy(zeros_c, num_s.at[pl.ds(k * CHUNK, CHUNK)])
      pltpu.sync_copy(zeros_c, den_s.at[pl.ds(k * CHUNK, CHUNK)])
    return 0

  lax.fori_loop(0, -(-(NP // CHUNK) // 16), _znd, 0, unroll=False)

  plsc.subcore_barrier()

  def _bucket(b_i, _):
    bkt = c * B_PER_CORE + b_i
    base_dst = bkt * C
    start = pl.multiple_of(bases[pl.ds(bkt * 16, 16)][0], 16)
    end = ctr3[pl.ds(bkt * 16, 16)][15]
    cnt = end - start
    nfull = cnt // CHUNK
    rem = cnt - nfull * CHUNK

    @pl.when(rem > 0)
    def _():
      def _tl(j, _):
        p = j * 16 + lane
        v = sorted_k[pl.ds(start + nfull * CHUNK + j * 16, 16)]
        valid = (nfull * CHUNK + p) < cnt
        tail_s[pl.ds(j * 16, 16)] = jnp.where(valid, v, DUMPBASE + p)
        return 0
      lax.fori_loop(0, CHUNK // 16, _tl, 0, unroll=False)

    def _att_into(slot):
      isdump = slot >= DUMPBASE
      srcv = slot % N
      dstv = jnp.where(isdump, 0, base_dst + slot // N)
      ssrc = plsc.load_gather(sc_scores, [srcv])
      sdst = plsc.load_gather(sc_scores, [dstv])
      return jnp.where(isdump, 0.0, a_c * ssrc + b_c * sdst + c_c)

    def _p1(k, _):
      pltpu.sync_copy(zeros_c, table.at[sorted_k.at[pl.ds(start + k * CHUNK,
                                                          CHUNK)]])
      return 0
    lax.fori_loop(0, nfull, _p1, 0, unroll=False)

    @pl.when(rem > 0)
    def _():
      pltpu.sync_copy(zeros_c, table.at[tail_s])
    plsc.subcore_barrier()

    def _p2(k, _):
      def _a(j, _):
        att_c[pl.ds(j * 16, 16)] = _att_into(
            sorted_k[pl.ds(start + k * CHUNK + j * 16, 16)])
        return 0
      lax.fori_loop(0, CHUNK // 16, _a, 0, unroll=False)
      pltpu.sync_copy(att_c, table.at[sorted_k.at[pl.ds(start + k * CHUNK,
                                                        CHUNK)]], add=True)
      return 0
    lax.fori_loop(0, nfull, _p2, 0, unroll=False)

    @pl.when(rem > 0)
    def _():
      def _a(j, _):
        att_c[pl.ds(j * 16, 16)] = _att_into(tail_s[pl.ds(j * 16, 16)])
        return 0
      lax.fori_loop(0, CHUNK // 16, _a, 0, unroll=False)
      pltpu.sync_copy(att_c, table.at[tail_s], add=True)
    plsc.subcore_barrier()

    def _p3(k, _):
      pltpu.sync_copy(table.at[sorted_k.at[pl.ds(start + k * CHUNK, CHUNK)]],
                      v_l.at[pl.ds(k * CHUNK, CHUNK)])
      return 0
    lax.fori_loop(0, nfull, _p3, 0, unroll=False)

    @pl.when(rem > 0)
    def _():
      pltpu.sync_copy(table.at[tail_s], v_l.at[pl.ds(nfull * CHUNK, CHUNK)])
    plsc.subcore_barrier()

    def _p4(k, _):
      pltpu.sync_copy(ones_c, table.at[sorted_k.at[pl.ds(start + k * CHUNK,
                                                         CHUNK)]], add=True)
      return 0
    lax.fori_loop(0, nfull, _p4, 0, unroll=False)

    @pl.when(rem > 0)
    def _():
      pltpu.sync_copy(ones_c, table.at[tail_s], add=True)
    plsc.subcore_barrier()

    def _p56_chunk(k, is_tail):
      if is_tail:
        pltpu.sync_copy(table.at[tail_s], vm_c)
      else:
        pltpu.sync_copy(table.at[sorted_k.at[pl.ds(start + k * CHUNK, CHUNK)]],
                        vm_c)

      def _inner(j, _):
        slc = pl.ds(j * 16, 16)
        if is_tail:
          slot = tail_s[slc]
          v = v_l[pl.ds(nfull * CHUNK + j * 16, 16)]
        else:
          slot = sorted_k[pl.ds(start + k * CHUNK + j * 16, 16)]
          v = v_l[pl.ds(k * CHUNK + j * 16, 16)]
        vm = vm_c[slc]
        mult = jnp.maximum((vm - v + 0.5).astype(jnp.int32)
                           .astype(jnp.float32), 1.0)
        lrel = jnp.where(v >= 0.0, v, 0.2 * v)
        w = (jnp.exp(lrel) - 1.0) / mult
        isdump = slot >= DUMPBASE
        srcv = slot % N
        dstv = jnp.where(isdump, 0, base_dst + slot // N)
        w = jnp.where(isdump, 0.0, w)
        sdst = plsc.load_gather(sc_scores, [dstv])
        srci_c[slc] = jnp.where(isdump, N + (slot - DUMPBASE) % 128, srcv)
        wnum_c[slc] = w * sdst
        wden_c[slc] = w
        return 0

      lax.fori_loop(0, CHUNK // 16, _inner, 0, unroll=False)
      d1 = pltpu.async_copy(wnum_c, num_s.at[srci_c], nd_sem1, add=True)
      d2 = pltpu.async_copy(wden_c, den_s.at[srci_c], nd_sem2, add=True)
      d1.wait()
      d2.wait()

    def _p56(k, _):
      _p56_chunk(k, False)
      return 0
    lax.fori_loop(0, nfull, _p56, 0, unroll=False)

    @pl.when(rem > 0)
    def _():
      _p56_chunk(0, True)
    plsc.subcore_barrier()
    return 0

  lax.fori_loop(0, B_PER_CORE, _bucket, 0, unroll=False)

  plsc.subcore_barrier()

  @pl.when(s == 0)
  def _():
    pltpu.sync_copy(num_s, num_hbm.at[c])
    pltpu.sync_copy(den_s, den_hbm.at[c])


def _edge_call(src, dst, scores, aux):
  mesh = plsc.VectorSubcoreMesh(core_axis_name="c", subcore_axis_name="s")
  f = pl.kernel(
      _edge_body,
      out_type=[
          jax.ShapeDtypeStruct((NSC, NP), jnp.float32),
          jax.ShapeDtypeStruct((NSC, NP), jnp.float32),
      ],
      mesh=mesh,
      compiler_params=pltpu.CompilerParams(use_tc_tiling_on_sc=False,
                                           needs_layout_passes=False),
      scratch_types=[
          pltpu.VMEM((N,), jnp.float32),
          pltpu.VMEM((SB,), jnp.int32),
          pltpu.VMEM((SB,), jnp.int32),
          pltpu.VMEM((LSZ,), jnp.int32),
          pltpu.VMEM((VSZ,), jnp.float32),
          pltpu.VMEM((HSZ,), jnp.int32),
          pltpu.VMEM((HSZ,), jnp.int32),
          pltpu.VMEM((HSZ,), jnp.int32),
          pltpu.VMEM((HSZ,), jnp.int32),
          pltpu.VMEM((HSZ,), jnp.int32),
          pltpu.VMEM((16,), jnp.float32),
          pltpu.VMEM((CHUNK,), jnp.float32),
          pltpu.VMEM((CHUNK,), jnp.float32),
          pltpu.VMEM((CHUNK,), jnp.float32),
          pltpu.VMEM((CHUNK,), jnp.int32),
          pltpu.VMEM((CHUNK,), jnp.float32),
          pltpu.VMEM((CHUNK,), jnp.float32),
          pltpu.VMEM((CHUNK,), jnp.float32),
          pltpu.VMEM((CHUNK,), jnp.int32),
          pltpu.SemaphoreType.DMA,
          pltpu.SemaphoreType.DMA,
          pltpu.VMEM_SHARED((NSLOT,), jnp.float32),
          pltpu.VMEM_SHARED((NP,), jnp.float32),
          pltpu.VMEM_SHARED((NP,), jnp.float32),
      ],
  )
  return f(src, dst, scores, aux)


def _final_body(num_ref, den_ref, aux_ref, out_ref):
  s_tot = aux_ref[0, 0]
  nsum = num_ref[0, :] + num_ref[1, :]
  dsum = den_ref[0, :] + den_ref[1, :]
  res = (s_tot + nsum) / (jnp.float32(N) + dsum)
  out_ref[...] = res[:, None]


def _final_call(num, den, aux):
  grid = 5
  r = NP // grid
  return pl.pallas_call(
      _final_body,
      grid=(grid,),
      in_specs=[
          pl.BlockSpec((2, r), lambda i: (0, i)),
          pl.BlockSpec((2, r), lambda i: (0, i)),
          pl.BlockSpec((1, 16), lambda i: (0, 0)),
      ],
      out_specs=pl.BlockSpec((r, 1), lambda i: (i, 0)),
      out_shape=jax.ShapeDtypeStruct((NP, 1), jnp.float32),
  )(num, den, aux)


@jax.jit
def kernel(x, edge_index, deg, W1, b1, W2, b2, edge_emb, att_w, gamma, beta):
  del deg, gamma, beta
  b1r = b1.reshape(1, H1)
  w2t = W2.reshape(1, H1)
  b2r = b2.reshape(1, 1)
  attwt = att_w.reshape(1, 22)
  scores2d, aux = _scores_call(x, W1, b1r, w2t, b2r, attwt, edge_emb)
  scores = scores2d.reshape(N)
  num, den = _edge_call(edge_index[0], edge_index[1], scores,
                        aux.reshape(16))
  return _final_call(num, den, aux)[:N]
